# Initial kernel scaffold; baseline (speedup 1.0000x reference)
import functools

import jax
import jax.numpy as jnp
from jax import lax
from jax.experimental import pallas as pl
from jax.experimental.pallas import tpu as pltpu
from jax.experimental.pallas import tpu_sc as plsc

N = 100000
D = 16
EPS = 0.01
PROP_STEP = 2
C_CONST = 0.01
E = 3200000

NPAD = 100096
CHUNK = 128
CPB = 50
BLK = CHUNK * CPB
TPW = 102400
NBLK = TPW // BLK
NWORKERS = 32
EPAD = TPW * NWORKERS
SID_ROWS = NPAD // 16

_mesh = plsc.VectorSubcoreMesh(core_axis_name="c", subcore_axis_name="s")


@functools.partial(
    pl.kernel,
    mesh=_mesh,
    out_type=jax.ShapeDtypeStruct((2, NPAD, D), jnp.float32),
    scratch_types=[
        pltpu.VMEM((CPB, CHUNK), jnp.int32),
        pltpu.VMEM((CPB, CHUNK), jnp.int32),
        pltpu.VMEM((CHUNK, D), jnp.float32),
        pltpu.VMEM((CHUNK, D), jnp.float32),
        pltpu.VMEM_SHARED((NPAD, D), jnp.float32),
        pltpu.SemaphoreType.DMA,
        pltpu.SemaphoreType.DMA,
    ],
)
def _sc_aggregate(y_hbm, src_hbm, dst_hbm, zeros_hbm, out_hbm,
                  srcb, dstb, rows_a, rows_b, acc, sem_a, sem_b):
    cid = lax.axis_index("c")
    sid = lax.axis_index("s")
    wid = cid * 16 + sid

    pltpu.sync_copy(zeros_hbm.at[pl.ds(sid * SID_ROWS, SID_ROWS)],
                    acc.at[pl.ds(sid * SID_ROWS, SID_ROWS)])
    plsc.subcore_barrier()

    base_row = wid * (TPW // CHUNK)

    def block_body(b, carry):
        row0 = base_row + b * CPB
        pltpu.sync_copy(src_hbm.at[pl.ds(row0, CPB)], srcb)
        pltpu.sync_copy(dst_hbm.at[pl.ds(row0, CPB)], dstb)

        def chunk_body(j, carry2):
            g_fwd = pltpu.async_copy(y_hbm.at[srcb.at[j]], rows_a, sem_a)
            g_bwd = pltpu.async_copy(y_hbm.at[dstb.at[j]], rows_b, sem_b)
            g_fwd.wait()
            pltpu.sync_copy(rows_a, acc.at[dstb.at[j]], add=True)
            g_bwd.wait()
            pltpu.sync_copy(rows_b, acc.at[srcb.at[j]], add=True)
            return carry2

        lax.fori_loop(0, CPB, chunk_body, 0)
        return carry

    lax.fori_loop(0, NBLK, block_body, 0)

    plsc.subcore_barrier()
    pltpu.sync_copy(acc.at[pl.ds(sid * SID_ROWS, SID_ROWS)],
                    out_hbm.at[cid, pl.ds(sid * SID_ROWS, SID_ROWS)])


_BU = 1564


def _tc_update_body(s0_ref, s1_ref, m8_ref, lab8_ref, rep_ref, h128_ref, y_ref):
    s = s0_ref[...] + s1_ref[...]
    rep = rep_ref[...]
    mrep = jnp.dot(m8_ref[...], rep, preferred_element_type=jnp.float32)
    labrep = jnp.dot(lab8_ref[...], rep, preferred_element_type=jnp.float32)
    cls = (lax.broadcasted_iota(jnp.int32, (_BU, 128), 1) % D).astype(jnp.float32)
    onehot = (labrep == cls).astype(jnp.float32)
    prior = mrep * ((D * C_CONST) * onehot - C_CONST)
    agg = jnp.dot(s, h128_ref[...], preferred_element_type=jnp.float32)
    y_ref[...] = prior + (EPS / D) * agg


_tc_update = pl.pallas_call(
    _tc_update_body,
    grid=(8,),
    in_specs=[
        pl.BlockSpec((_BU, 128), lambda i: (i, 0)),
        pl.BlockSpec((_BU, 128), lambda i: (i, 0)),
        pl.BlockSpec((_BU, 8), lambda i: (i, 0)),
        pl.BlockSpec((_BU, 8), lambda i: (i, 0)),
        pl.BlockSpec((8, 128), lambda i: (0, 0)),
        pl.BlockSpec((128, 128), lambda i: (0, 0)),
    ],
    out_specs=pl.BlockSpec((_BU, 128), lambda i: (i, 0)),
    out_shape=jax.ShapeDtypeStruct((NPAD // 8, 128), jnp.float32),
)


def kernel(X, H, edge_index, train_mask, labels):
    src = edge_index[0]
    dst = edge_index[1]
    pad_idx = jnp.full((EPAD - E,), N, dtype=jnp.int32)
    src2d = jnp.concatenate([src, pad_idx]).reshape(EPAD // CHUNK, CHUNK)
    dst2d = jnp.concatenate([dst, pad_idx]).reshape(EPAD // CHUNK, CHUNK)

    m8 = jnp.pad(train_mask.astype(jnp.float32), (0, NPAD - N)).reshape(NPAD // 8, 8)
    lab8 = jnp.pad(labels.astype(jnp.float32), (0, NPAD - N)).reshape(NPAD // 8, 8)

    rep = jnp.repeat(jnp.eye(8, dtype=jnp.float32), D, axis=1)
    h128 = jnp.kron(jnp.eye(8, dtype=jnp.float32), H.astype(jnp.float32))

    zeros_rows = jnp.zeros((NPAD, D), jnp.float32)
    zeros_packed = jnp.zeros((NPAD // 8, 128), jnp.float32)

    y = _tc_update(zeros_packed, zeros_packed, m8, lab8, rep, h128)
    for _ in range(PROP_STEP):
        s_parts = _sc_aggregate(y.reshape(NPAD, D), src2d, dst2d, zeros_rows)
        s0 = s_parts[0].reshape(NPAD // 8, 128)
        s1 = s_parts[1].reshape(NPAD // 8, 128)
        y = _tc_update(s0, s1, m8, lab8, rep, h128)
    return y.reshape(NPAD, D)[:N]


# speedup vs baseline: 18.7929x; 18.7929x over previous
import functools

import jax
import jax.numpy as jnp
from jax import lax
from jax.experimental import pallas as pl
from jax.experimental.pallas import tpu as pltpu
from jax.experimental.pallas import tpu_sc as plsc

N = 100000
D = 16
EPS = 0.01
PROP_STEP = 2
C_CONST = 0.01
E = 3200000

NPAD = 100096
CHUNK = 128
CPB = 80
BLK = CHUNK * CPB
TPW = 102400
NBLK = TPW // BLK
NWORKERS = 32
EPAD = TPW * NWORKERS
SID_ROWS = NPAD // 16

_mesh = plsc.VectorSubcoreMesh(core_axis_name="c", subcore_axis_name="s")


@functools.partial(
    pl.kernel,
    mesh=_mesh,
    compiler_params=pltpu.CompilerParams(use_tc_tiling_on_sc=False),
    out_type=jax.ShapeDtypeStruct((2, NPAD, D), jnp.float32),
    scratch_types=[
        pltpu.VMEM((CPB, CHUNK), jnp.int32),
        pltpu.VMEM((CPB, CHUNK), jnp.int32),
        pltpu.VMEM((CHUNK, D), jnp.float32),
        pltpu.VMEM((CHUNK, D), jnp.float32),
        pltpu.VMEM_SHARED((NPAD, D), jnp.float32),
        pltpu.SemaphoreType.DMA,
        pltpu.SemaphoreType.DMA,
    ],
)
def _sc_aggregate(y_hbm, src_hbm, dst_hbm, zeros_hbm, out_hbm,
                  srcb, dstb, rows_a, rows_b, acc, sem_a, sem_b):
    cid = lax.axis_index("c")
    sid = lax.axis_index("s")
    wid = cid * 16 + sid

    pltpu.sync_copy(zeros_hbm.at[pl.ds(sid * SID_ROWS, SID_ROWS)],
                    acc.at[pl.ds(sid * SID_ROWS, SID_ROWS)])
    plsc.subcore_barrier()

    base_row = wid * (TPW // CHUNK)

    def block_body(b, carry):
        row0 = base_row + b * CPB
        pltpu.sync_copy(src_hbm.at[pl.ds(row0, CPB)], srcb)
        pltpu.sync_copy(dst_hbm.at[pl.ds(row0, CPB)], dstb)

        def chunk_body(j, carry2):
            g_fwd = pltpu.async_copy(y_hbm.at[srcb.at[j]], rows_a, sem_a)
            g_bwd = pltpu.async_copy(y_hbm.at[dstb.at[j]], rows_b, sem_b)
            g_fwd.wait()
            pltpu.sync_copy(rows_a, acc.at[dstb.at[j]], add=True)
            g_bwd.wait()
            pltpu.sync_copy(rows_b, acc.at[srcb.at[j]], add=True)
            return carry2

        lax.fori_loop(0, CPB, chunk_body, 0)
        return carry

    lax.fori_loop(0, NBLK, block_body, 0)

    plsc.subcore_barrier()
    pltpu.sync_copy(acc.at[pl.ds(sid * SID_ROWS, SID_ROWS)],
                    out_hbm.at[cid, pl.ds(sid * SID_ROWS, SID_ROWS)])


_BU = 3128


def _tc_update_body(s0_ref, s1_ref, m8_ref, lab8_ref, rep_ref, h128_ref, y_ref):
    s = s0_ref[...] + s1_ref[...]
    rep = rep_ref[...]
    mrep = jnp.dot(m8_ref[...], rep, preferred_element_type=jnp.float32)
    labrep = jnp.dot(lab8_ref[...], rep, preferred_element_type=jnp.float32)
    cls = (lax.broadcasted_iota(jnp.int32, (_BU, 128), 1) % D).astype(jnp.float32)
    onehot = (labrep == cls).astype(jnp.float32)
    prior = mrep * ((D * C_CONST) * onehot - C_CONST)
    agg = jnp.dot(s, h128_ref[...], preferred_element_type=jnp.float32)
    y_ref[...] = prior + (EPS / D) * agg


_tc_update = pl.pallas_call(
    _tc_update_body,
    grid=(4,),
    in_specs=[
        pl.BlockSpec((_BU, 128), lambda i: (i, 0)),
        pl.BlockSpec((_BU, 128), lambda i: (i, 0)),
        pl.BlockSpec((_BU, 8), lambda i: (i, 0)),
        pl.BlockSpec((_BU, 8), lambda i: (i, 0)),
        pl.BlockSpec((8, 128), lambda i: (0, 0)),
        pl.BlockSpec((128, 128), lambda i: (0, 0)),
    ],
    out_specs=pl.BlockSpec((_BU, 128), lambda i: (i, 0)),
    out_shape=jax.ShapeDtypeStruct((NPAD // 8, 128), jnp.float32),
)


def kernel(X, H, edge_index, train_mask, labels):
    src = edge_index[0]
    dst = edge_index[1]
    pad_idx = jnp.full((EPAD - E,), N, dtype=jnp.int32)
    src2d = jnp.concatenate([src, pad_idx]).reshape(EPAD // CHUNK, CHUNK)
    dst2d = jnp.concatenate([dst, pad_idx]).reshape(EPAD // CHUNK, CHUNK)

    m8 = jnp.pad(train_mask.astype(jnp.float32), (0, NPAD - N)).reshape(NPAD // 8, 8)
    lab8 = jnp.pad(labels.astype(jnp.float32), (0, NPAD - N)).reshape(NPAD // 8, 8)

    rep = jnp.repeat(jnp.eye(8, dtype=jnp.float32), D, axis=1)
    h128 = jnp.kron(jnp.eye(8, dtype=jnp.float32), H.astype(jnp.float32))

    zeros_rows = jnp.zeros((NPAD, D), jnp.float32)
    zeros_packed = jnp.zeros((NPAD // 8, 128), jnp.float32)

    y = _tc_update(zeros_packed, zeros_packed, m8, lab8, rep, h128)
    for _ in range(PROP_STEP):
        s_parts = _sc_aggregate(y.reshape(NPAD, D), src2d, dst2d, zeros_rows)
        s0 = s_parts[0].reshape(NPAD // 8, 128)
        s1 = s_parts[1].reshape(NPAD // 8, 128)
        y = _tc_update(s0, s1, m8, lab8, rep, h128)
    return y.reshape(NPAD, D)[:N]


# speedup vs baseline: 66.1614x; 3.5206x over previous
import functools

import jax
import jax.numpy as jnp
from jax import lax
from jax.experimental import pallas as pl
from jax.experimental.pallas import tpu as pltpu
from jax.experimental.pallas import tpu_sc as plsc

N = 100000
D = 16
EPS = 0.01
PROP_STEP = 2
C_CONST = 0.01
E = 3200000

NPAD = 100096
CHUNK = 128
ECH = E // CHUNK
CPB = 78
NBLK = 10
MAIN_ROWS = 32 * CPB * NBLK
XTRA = (ECH - MAIN_ROWS) // 8
SID_ROWS = NPAD // 16
PACK_ROWS = NPAD // 8

_mesh = plsc.VectorSubcoreMesh(core_axis_name="c", subcore_axis_name="s")


@functools.partial(
    pl.kernel,
    mesh=_mesh,
    compiler_params=pltpu.CompilerParams(use_tc_tiling_on_sc=False),
    out_type=jax.ShapeDtypeStruct((2, NPAD, D), jnp.float32),
    scratch_types=[
        pltpu.VMEM((CPB, CHUNK), jnp.int32),
        pltpu.VMEM((CPB, CHUNK), jnp.int32),
        pltpu.VMEM((2, CHUNK, D), jnp.float32),
        pltpu.VMEM((2, CHUNK, D), jnp.float32),
        pltpu.VMEM_SHARED((NPAD, D), jnp.float32),
        pltpu.SemaphoreType.DMA((2,)),
        pltpu.SemaphoreType.DMA((2,)),
        pltpu.SemaphoreType.DMA((2,)),
        pltpu.SemaphoreType.DMA((2,)),
    ],
)
def _sc_aggregate(y_hbm, edges_hbm, zeros_hbm, out_hbm,
                  srcb, dstb, rows_f, rows_b, acc,
                  sem_gf, sem_gb, sem_sf, sem_sb):
    cid = lax.axis_index("c")
    sid = lax.axis_index("s")
    wid = cid * 16 + sid

    pltpu.sync_copy(zeros_hbm.at[pl.ds(sid * SID_ROWS, SID_ROWS)],
                    acc.at[pl.ds(sid * SID_ROWS, SID_ROWS)])
    plsc.subcore_barrier()

    def gather(j, slot):
        pltpu.async_copy(y_hbm.at[srcb.at[j]], rows_f.at[slot], sem_gf.at[slot])
        pltpu.async_copy(y_hbm.at[dstb.at[j]], rows_b.at[slot], sem_gb.at[slot])

    def wait_gather(j, slot):
        pltpu.make_async_copy(y_hbm.at[srcb.at[j]], rows_f.at[slot], sem_gf.at[slot]).wait()
        pltpu.make_async_copy(y_hbm.at[dstb.at[j]], rows_b.at[slot], sem_gb.at[slot]).wait()

    def scatter(j, slot):
        pltpu.async_copy(rows_f.at[slot], acc.at[dstb.at[j]], sem_sf.at[slot], add=True)
        pltpu.async_copy(rows_b.at[slot], acc.at[srcb.at[j]], sem_sb.at[slot], add=True)

    def wait_scatter(slot):
        pltpu.make_async_copy(rows_f.at[slot], acc.at[dstb.at[0]], sem_sf.at[slot]).wait()
        pltpu.make_async_copy(rows_b.at[slot], acc.at[srcb.at[0]], sem_sb.at[slot]).wait()

    base_row = wid * CPB * NBLK

    def block_body(b, carry):
        row0 = base_row + b * CPB
        pltpu.sync_copy(edges_hbm.at[pl.ds(row0, CPB)], srcb)
        pltpu.sync_copy(edges_hbm.at[pl.ds(ECH + row0, CPB)], dstb)

        gather(0, 0)

        def pair_body(p, carry2):
            for par in (0, 1):
                j = 2 * p + par
                if par == 0:
                    @pl.when(p >= 1)
                    def _():
                        wait_scatter(1)
                        gather(j + 1, 1)

                    @pl.when(p == 0)
                    def _():
                        gather(j + 1, 1)
                else:
                    @pl.when(p <= CPB // 2 - 2)
                    def _():
                        wait_scatter(0)
                        gather(j + 1, 0)
                wait_gather(j, par)
                scatter(j, par)
            return carry2

        lax.fori_loop(0, CPB // 2, pair_body, 0)
        wait_scatter(0)
        wait_scatter(1)
        return carry

    lax.fori_loop(0, NBLK, block_body, 0)

    @pl.when(wid < 8)
    def _():
        pltpu.sync_copy(edges_hbm.at[pl.ds(MAIN_ROWS + wid * XTRA, XTRA)],
                        srcb.at[pl.ds(0, XTRA)])
        pltpu.sync_copy(edges_hbm.at[pl.ds(ECH + MAIN_ROWS + wid * XTRA, XTRA)],
                        dstb.at[pl.ds(0, XTRA)])

        def xtra_body(j, carry):
            gather(j, 0)
            wait_gather(j, 0)
            scatter(j, 0)
            wait_scatter(0)
            return carry

        lax.fori_loop(0, XTRA, xtra_body, 0)

    plsc.subcore_barrier()
    pltpu.sync_copy(acc.at[pl.ds(sid * SID_ROWS, SID_ROWS)],
                    out_hbm.at[cid, pl.ds(sid * SID_ROWS, SID_ROWS)])


_BU = 3128
_SOFF = PACK_ROWS // _BU


def _tc_update_body(s0_ref, s1_ref, m8_ref, lab8_ref, rep_ref, h128_ref, y_ref):
    s = s0_ref[...] + s1_ref[...]
    rep = rep_ref[...]
    mrep = jnp.dot(m8_ref[...], rep, preferred_element_type=jnp.float32)
    labrep = jnp.dot(lab8_ref[...], rep, preferred_element_type=jnp.float32)
    cls = (lax.broadcasted_iota(jnp.int32, (_BU, 128), 1) % D).astype(jnp.float32)
    onehot = (labrep == cls).astype(jnp.float32)
    prior = mrep * ((D * C_CONST) * onehot - C_CONST)
    agg = jnp.dot(s, h128_ref[...], preferred_element_type=jnp.float32)
    y_ref[...] = prior + (EPS / D) * agg


def _make_tc_update(dup_input):
    s1_map = (lambda i: (i + _SOFF, 0)) if dup_input else (lambda i: (i, 0))
    return pl.pallas_call(
        _tc_update_body,
        grid=(PACK_ROWS // _BU,),
        in_specs=[
            pl.BlockSpec((_BU, 128), lambda i: (i, 0)),
            pl.BlockSpec((_BU, 128), s1_map),
            pl.BlockSpec((_BU, 8), lambda i: (i, 0)),
            pl.BlockSpec((_BU, 8), lambda i: (i, 0)),
            pl.BlockSpec((8, 128), lambda i: (0, 0)),
            pl.BlockSpec((128, 128), lambda i: (0, 0)),
        ],
        out_specs=pl.BlockSpec((_BU, 128), lambda i: (i, 0)),
        out_shape=jax.ShapeDtypeStruct((PACK_ROWS, 128), jnp.float32),
    )


_tc_update_pair = _make_tc_update(True)
_tc_update_zero = _make_tc_update(False)


def kernel(X, H, edge_index, train_mask, labels):
    m8 = jnp.pad(train_mask.astype(jnp.float32), (0, NPAD - N)).reshape(PACK_ROWS, 8)
    lab8 = jnp.pad(labels.astype(jnp.float32), (0, NPAD - N)).reshape(PACK_ROWS, 8)

    rep = jnp.repeat(jnp.eye(8, dtype=jnp.float32), D, axis=1)
    h128 = jnp.kron(jnp.eye(8, dtype=jnp.float32), H.astype(jnp.float32))

    edges = edge_index.reshape(2 * ECH, CHUNK)
    zeros_rows = jnp.zeros((NPAD, D), jnp.float32)
    zeros_packed = jnp.zeros((PACK_ROWS, 128), jnp.float32)

    y = _tc_update_zero(zeros_packed, zeros_packed, m8, lab8, rep, h128)
    for _ in range(PROP_STEP):
        s_pair = _sc_aggregate(y.reshape(NPAD, D), edges, zeros_rows)
        s_flat = s_pair.reshape(2 * PACK_ROWS, 128)
        y = _tc_update_pair(s_flat, s_flat, m8, lab8, rep, h128)
    return y.reshape(NPAD, D)[:N]


# speedup vs baseline: 99.6995x; 1.5069x over previous
import functools

import jax
import jax.numpy as jnp
from jax import lax
from jax.experimental import pallas as pl
from jax.experimental.pallas import tpu as pltpu
from jax.experimental.pallas import tpu_sc as plsc

N = 100000
D = 16
EPS = 0.01
PROP_STEP = 2
C_CONST = 0.01
E = 3200000

NPAD = 100096
CHUNK = 128
ECH = E // CHUNK
IDXB = 20
NBLK = 39
NCH = IDXB * NBLK
NG = NCH // 4
MAIN_ROWS = 32 * NCH
XTRA = (ECH - MAIN_ROWS) // 8
SID_ROWS = NPAD // 16
PACK_ROWS = NPAD // 8

_mesh = plsc.VectorSubcoreMesh(core_axis_name="c", subcore_axis_name="s")


@functools.partial(
    pl.kernel,
    mesh=_mesh,
    compiler_params=pltpu.CompilerParams(use_tc_tiling_on_sc=False),
    out_type=jax.ShapeDtypeStruct((2, NPAD, D), jnp.float32),
    scratch_types=[
        pltpu.VMEM((2, IDXB, CHUNK), jnp.int32),
        pltpu.VMEM((2, IDXB, CHUNK), jnp.int32),
        pltpu.VMEM((4, CHUNK, D), jnp.float32),
        pltpu.VMEM((4, CHUNK, D), jnp.float32),
        pltpu.VMEM_SHARED((NPAD, D), jnp.float32),
        pltpu.SemaphoreType.DMA((4,)),
        pltpu.SemaphoreType.DMA((4,)),
        pltpu.SemaphoreType.DMA((4,)),
        pltpu.SemaphoreType.DMA((4,)),
        pltpu.SemaphoreType.DMA((2,)),
    ],
)
def _sc_aggregate(y_hbm, edges_hbm, zeros_hbm, out_hbm,
                  srcb, dstb, rows_f, rows_b, acc,
                  sem_gf, sem_gb, sem_sf, sem_sb, sem_i):
    cid = lax.axis_index("c")
    sid = lax.axis_index("s")
    wid = cid * 16 + sid

    pltpu.sync_copy(zeros_hbm.at[pl.ds(sid * SID_ROWS, SID_ROWS)],
                    acc.at[pl.ds(sid * SID_ROWS, SID_ROWS)])
    plsc.subcore_barrier()

    def gather(h, row, slot):
        pltpu.async_copy(y_hbm.at[srcb.at[h, row]], rows_f.at[slot], sem_gf.at[slot])
        pltpu.async_copy(y_hbm.at[dstb.at[h, row]], rows_b.at[slot], sem_gb.at[slot])

    def wait_gather(slot):
        pltpu.make_async_copy(y_hbm.at[srcb.at[0, 0]], rows_f.at[slot], sem_gf.at[slot]).wait()
        pltpu.make_async_copy(y_hbm.at[dstb.at[0, 0]], rows_b.at[slot], sem_gb.at[slot]).wait()

    def scatter(h, row, slot):
        pltpu.async_copy(rows_f.at[slot], acc.at[dstb.at[h, row]], sem_sf.at[slot], add=True)
        pltpu.async_copy(rows_b.at[slot], acc.at[srcb.at[h, row]], sem_sb.at[slot], add=True)

    def wait_scatter(slot):
        pltpu.make_async_copy(rows_f.at[slot], acc.at[dstb.at[0, 0]], sem_sf.at[slot]).wait()
        pltpu.make_async_copy(rows_b.at[slot], acc.at[srcb.at[0, 0]], sem_sb.at[slot]).wait()

    base_row = wid * NCH

    def idx_load(blk, half):
        row0 = base_row + blk * IDXB
        pltpu.async_copy(edges_hbm.at[pl.ds(row0, IDXB)], srcb.at[half], sem_i.at[half])
        pltpu.async_copy(edges_hbm.at[pl.ds(ECH + row0, IDXB)], dstb.at[half], sem_i.at[half])

    def wait_idx(half):
        pltpu.make_async_copy(edges_hbm.at[pl.ds(0, IDXB)], srcb.at[half], sem_i.at[half]).wait()
        pltpu.make_async_copy(edges_hbm.at[pl.ds(0, IDXB)], dstb.at[half], sem_i.at[half]).wait()

    idx_load(0, 0)
    idx_load(1, 1)
    wait_idx(0)
    gather(0, 0, 0)
    gather(0, 1, 1)

    def group_body(g, carry):
        j0 = 4 * g
        q0 = j0 // IDXB
        r0 = j0 - q0 * IDXB
        h0 = lax.rem(q0, 2)

        def coords(off):
            rr = r0 + off
            wrap = jnp.where(rr >= IDXB, 1, 0)
            return lax.rem(q0 + wrap, 2), rr - IDXB * wrap

        @pl.when((r0 == 8) & (q0 >= 1) & (q0 + 1 < NBLK))
        def _():
            idx_load(q0 + 1, lax.rem(q0 + 1, 2))

        @pl.when((r0 == IDXB - 4) & (q0 + 1 < NBLK))
        def _():
            wait_idx(lax.rem(q0 + 1, 2))

        for par in range(4):
            j = j0 + par
            s_cur = par & 3
            s_nxt = (par + 2) & 3
            h2, row2 = coords(par + 2)

            def refill(do_drain, s_nxt=s_nxt, h2=h2, row2=row2):
                if do_drain:
                    wait_scatter(s_nxt)
                gather(h2, row2, s_nxt)

            if par < 2:
                @pl.when(g >= 1)
                def _():
                    refill(True)

                @pl.when(g == 0)
                def _():
                    refill(False)
            else:
                @pl.when(g <= NG - 2)
                def _():
                    refill(True)
            h, row = coords(par)
            wait_gather(s_cur)
            scatter(h, row, s_cur)
        return carry

    lax.fori_loop(0, NG, group_body, 0)
    for slot in range(4):
        wait_scatter(slot)

    @pl.when(wid < 8)
    def _():
        pltpu.sync_copy(edges_hbm.at[pl.ds(MAIN_ROWS + wid * XTRA, XTRA)],
                        srcb.at[0, pl.ds(0, XTRA)])
        pltpu.sync_copy(edges_hbm.at[pl.ds(ECH + MAIN_ROWS + wid * XTRA, XTRA)],
                        dstb.at[0, pl.ds(0, XTRA)])

        def xtra_body(j, carry):
            gather(0, j, 0)
            wait_gather(0)
            scatter(0, j, 0)
            wait_scatter(0)
            return carry

        lax.fori_loop(0, XTRA, xtra_body, 0)

    plsc.subcore_barrier()
    pltpu.sync_copy(acc.at[pl.ds(sid * SID_ROWS, SID_ROWS)],
                    out_hbm.at[cid, pl.ds(sid * SID_ROWS, SID_ROWS)])


_BU = 3128
_SOFF = PACK_ROWS // _BU


def _tc_update_body(s0_ref, s1_ref, m8_ref, lab8_ref, rep_ref, h128_ref, y_ref):
    s = s0_ref[...] + s1_ref[...]
    rep = rep_ref[...]
    mrep = jnp.dot(m8_ref[...], rep, preferred_element_type=jnp.float32)
    labrep = jnp.dot(lab8_ref[...], rep, preferred_element_type=jnp.float32)
    cls = (lax.broadcasted_iota(jnp.int32, (_BU, 128), 1) % D).astype(jnp.float32)
    onehot = (labrep == cls).astype(jnp.float32)
    prior = mrep * ((D * C_CONST) * onehot - C_CONST)
    agg = jnp.dot(s, h128_ref[...], preferred_element_type=jnp.float32)
    y_ref[...] = prior + (EPS / D) * agg


def _make_tc_update(dup_input):
    s1_map = (lambda i: (i + _SOFF, 0)) if dup_input else (lambda i: (i, 0))
    return pl.pallas_call(
        _tc_update_body,
        grid=(PACK_ROWS // _BU,),
        in_specs=[
            pl.BlockSpec((_BU, 128), lambda i: (i, 0)),
            pl.BlockSpec((_BU, 128), s1_map),
            pl.BlockSpec((_BU, 8), lambda i: (i, 0)),
            pl.BlockSpec((_BU, 8), lambda i: (i, 0)),
            pl.BlockSpec((8, 128), lambda i: (0, 0)),
            pl.BlockSpec((128, 128), lambda i: (0, 0)),
        ],
        out_specs=pl.BlockSpec((_BU, 128), lambda i: (i, 0)),
        out_shape=jax.ShapeDtypeStruct((PACK_ROWS, 128), jnp.float32),
    )


_tc_update_pair = _make_tc_update(True)
_tc_update_zero = _make_tc_update(False)


def kernel(X, H, edge_index, train_mask, labels):
    m8 = jnp.pad(train_mask.astype(jnp.float32), (0, NPAD - N)).reshape(PACK_ROWS, 8)
    lab8 = jnp.pad(labels.astype(jnp.float32), (0, NPAD - N)).reshape(PACK_ROWS, 8)

    rep = jnp.repeat(jnp.eye(8, dtype=jnp.float32), D, axis=1)
    h128 = jnp.kron(jnp.eye(8, dtype=jnp.float32), H.astype(jnp.float32))

    edges = edge_index.reshape(2 * ECH, CHUNK)
    zeros_rows = jnp.zeros((NPAD, D), jnp.float32)
    zeros_packed = jnp.zeros((PACK_ROWS, 128), jnp.float32)

    y = _tc_update_zero(zeros_packed, zeros_packed, m8, lab8, rep, h128)
    for _ in range(PROP_STEP):
        s_pair = _sc_aggregate(y.reshape(NPAD, D), edges, zeros_rows)
        s_flat = s_pair.reshape(2 * PACK_ROWS, 128)
        y = _tc_update_pair(s_flat, s_flat, m8, lab8, rep, h128)
    return y.reshape(NPAD, D)[:N]
